# baseline (speedup 1.0000x reference)
import jax
import jax.numpy as jnp
from jax.experimental import pallas as pl
from jax.experimental.pallas import tpu as pltpu

_EPS = 1e-5
_B, _C, _H, _W = 32, 3, 400, 600
_KS = 5
_HO, _WO, _OC = _H // _KS, _W // _KS, 16
_HT = 16
_RT = _HT * _KS
_CR = _C * _RT
_NHT = _HO // _HT
_WP = 128


def _fused_kernel(x_ref, wr_ref, s3_ref, lc_ref, b_ref, whc_ref, o_ref):
    def tile_body(ht, carry):
        a0, a1 = carry
        r0 = ht * _RT
        h0 = ht * _HT
        xbf = jnp.concatenate(
            [x_ref[0, c, pl.ds(r0, _RT), :].astype(jnp.bfloat16)
             for c in range(_C)], axis=0)
        for oc in range(_OC):
            t = xbf * wr_ref[oc]
            pre = jnp.dot(s3_ref[...], t,
                          preferred_element_type=jnp.float32)
            z = jnp.dot(pre, lc_ref[...],
                        preferred_element_type=jnp.float32)
            z = jnp.maximum(z + b_ref[oc], 0.0)
            a0 = a0 + jnp.sum(z * whc_ref[0, oc, pl.ds(h0, _HT), :],
                              axis=0, keepdims=True)
            a1 = a1 + jnp.sum(z * whc_ref[1, oc, pl.ds(h0, _HT), :],
                              axis=0, keepdims=True)
        return (a0, a1)

    zero = jnp.zeros((1, _WP), jnp.float32)
    a0, a1 = jax.lax.fori_loop(0, _NHT, tile_body, (zero, zero))
    t0 = jnp.sum(a0)
    t1 = jnp.sum(a1)
    lane = jax.lax.broadcasted_iota(jnp.int32, (1, 1, 128), 2)
    o_ref[...] = jnp.where(lane == 0, t0, jnp.where(lane == 1, t1, 0.0))


def kernel(x, conv_w, conv_b, bn_gamma, bn_beta, bn_mean, bn_var,
           head_w, head_b):
    bn_scale = bn_gamma * jax.lax.rsqrt(bn_var + _EPS)
    w_sc = conv_w * bn_scale[:, None, None, None]
    b_eff = bn_scale * (conv_b - bn_mean) + bn_beta

    wr = jnp.tile(w_sc, (1, 1, _RT // _KS, _WO))
    wr = wr.reshape(_OC, _CR, _W).astype(jnp.bfloat16)

    rr = jax.lax.broadcasted_iota(jnp.int32, (_HT, _RT), 1)
    hh = jax.lax.broadcasted_iota(jnp.int32, (_HT, _RT), 0)
    s_band = ((rr >= _KS * hh) & (rr < _KS * hh + _KS))
    s3 = jnp.tile(s_band, (1, _C)).astype(jnp.bfloat16)

    ll = jax.lax.broadcasted_iota(jnp.int32, (_W, _WP), 0)
    ww = jax.lax.broadcasted_iota(jnp.int32, (_W, _WP), 1)
    lc = ((ll >= _KS * ww) & (ll < _KS * ww + _KS)
          & (ww < _WO)).astype(jnp.float32)

    wh = head_w.reshape(2, _OC, _HO, _WO)
    whc = jnp.pad(wh, ((0, 0), (0, 0), (0, 0), (0, _WP - _WO)))

    out_pad = pl.pallas_call(
        _fused_kernel,
        out_shape=jax.ShapeDtypeStruct((_B, 1, 128), jnp.float32),
        grid_spec=pltpu.PrefetchScalarGridSpec(
            num_scalar_prefetch=0,
            grid=(_B,),
            in_specs=[
                pl.BlockSpec((1, _C, _H, _W), lambda b: (b, 0, 0, 0)),
                pl.BlockSpec((_OC, _CR, _W), lambda b: (0, 0, 0)),
                pl.BlockSpec((_HT, _CR), lambda b: (0, 0)),
                pl.BlockSpec((_W, _WP), lambda b: (0, 0)),
                pl.BlockSpec(memory_space=pltpu.SMEM),
                pl.BlockSpec((2, _OC, _HO, _WP), lambda b: (0, 0, 0, 0)),
            ],
            out_specs=pl.BlockSpec((1, 1, 128), lambda b: (b, 0, 0)),
        ),
        compiler_params=pltpu.CompilerParams(
            dimension_semantics=("parallel",)),
    )(x, wr, s3, lc, b_eff, whc)

    return out_pad[:, 0, :2] + head_b[None, :]
